# baseline (speedup 1.0000x reference)
import functools

import jax
import jax.numpy as jnp
from jax import lax
from jax.experimental import pallas as pl
from jax.experimental.pallas import tpu as pltpu
from jax.experimental.pallas import tpu_sc as plsc

_L = 512
_C = 128
_OUT = 128
_K = 10
_NW = 32
_RPW = 4 * _L // _NW
_LANES = 16
_NCH = _L // _LANES
_QW = _RPW * _C


def _tc_body(x_ref, w0_ref, w1_ref, w2_ref, bc_ref, wln_ref, wlc_ref, bl_ref,
             d0_ref, d1_ref, d2_ref_o, d3_ref, pcb_ref, ct_ref, tbl_ref,
             d2rs_ref):
    xb = x_ref[0]
    hi = jax.lax.Precision.HIGHEST
    dg = functools.partial(jax.lax.dot_general, precision=hi,
                           preferred_element_type=jnp.float32)

    G = dg(xb, xb, (((0,), (0,)), ((), ())))
    d2 = jnp.sum(xb * xb, axis=0)
    rs = jax.lax.rsqrt(d2)
    dif = d2[None, :] + d2[:, None] - 2.0 * G
    d0_ref[0] = dif[:, 0 * _C:1 * _C]
    d1_ref[0] = dif[:, 1 * _C:2 * _C]
    d2_ref_o[0] = dif[:, 2 * _C:3 * _C]
    d3_ref[0] = dif[:, 3 * _C:4 * _C]
    d2rs_ref[0, 0] = jnp.concatenate([d2, rs], axis=0)

    zcol = jnp.zeros((_C, 1), jnp.float32)
    xl = jnp.concatenate([xb[:, 1:], zcol], axis=1)
    xr = jnp.concatenate([zcol, xb[:, :-1]], axis=1)
    ct_ref[0] = (dg(xr, w0_ref[...], (((0,), (1,)), ((), ())))
                 + dg(xb, w1_ref[...], (((0,), (1,)), ((), ())))
                 + dg(xl, w2_ref[...], (((0,), (1,)), ((), ())))
                 + bc_ref[...])

    tbl_ref[0] = dg(xb, wln_ref[...], (((0,), (1,)), ((), ())))
    pcb_ref[0] = dg(xb, wlc_ref[...], (((0,), (1,)), ((), ()))) + bl_ref[...]


def _tc_stage(x, w0, w1, w2, bc, wln, wlc, bl):
    bs = x.shape[0]
    full = lambda s: pl.BlockSpec(s, lambda b: (0,) * len(s))
    plane = pl.BlockSpec((1, _L, _C), lambda b: (b, 0, 0))
    pshape = jax.ShapeDtypeStruct((bs, _L, _C), jnp.float32)
    return pl.pallas_call(
        _tc_body,
        grid=(bs,),
        in_specs=[
            pl.BlockSpec((1, _C, _L), lambda b: (b, 0, 0)),
            full((_OUT, _C)), full((_OUT, _C)), full((_OUT, _C)),
            full((1, _OUT)),
            full((_OUT, _C)), full((_OUT, _C)),
            full((1, _OUT)),
        ],
        out_specs=[plane] * 7 + [pl.BlockSpec((1, 1, 2 * _L),
                                              lambda b: (b, 0, 0))],
        out_shape=[pshape] * 7 + [jax.ShapeDtypeStruct((bs, 1, 2 * _L),
                                                       jnp.float32)],
    )(x, w0, w1, w2, bc, wln, wlc, bl)


def _topk16(difb, r, jidx):
    def merge(a, b, desc):
        ka, va = a
        kb, vb = b
        take = ka <= kb
        mk = jnp.where(take, ka, kb)
        mv = jnp.where(take, va, vb)
        return plsc.sort_key_val(mk, mv, descending=desc)

    def tree(lo, hi, desc):
        if hi - lo == 1:
            off = (lo // 8) * _QW + r * _C + (lo % 8) * _LANES
            k = difb[pl.ds(off, _LANES)]
            return plsc.sort_key_val(k, jidx + lo * _LANES, descending=desc)
        mid = (lo + hi) // 2
        return merge(tree(lo, mid, False), tree(mid, hi, True), desc)

    return tree(0, _NCH, False)


def _sc_stage(d0, d1, d2f, d3, pcbf, ctf, tbl, d2rs_f):
    mesh = plsc.VectorSubcoreMesh(core_axis_name="c", subcore_axis_name="s")

    @functools.partial(
        pl.kernel,
        mesh=mesh,
        compiler_params=pltpu.CompilerParams(needs_layout_passes=False),
        out_type=jax.ShapeDtypeStruct((4 * _L // 2 * _OUT,), jnp.float32),
        scratch_types=[
            pltpu.VMEM((4 * _QW,), jnp.float32),
            pltpu.VMEM((_RPW * _OUT,), jnp.float32),
            pltpu.VMEM((_RPW * _OUT,), jnp.float32),
            pltpu.VMEM((_L, _OUT), jnp.float32),
            pltpu.VMEM((_L,), jnp.float32),
            pltpu.VMEM((_L,), jnp.float32),
            pltpu.VMEM((_RPW // 2 * _OUT,), jnp.float32),
            pltpu.VMEM((_RPW * _LANES,), jnp.float32),
            pltpu.VMEM((_RPW * _LANES,), jnp.int32),
            pltpu.SemaphoreType.DMA,
            pltpu.SemaphoreType.DMA,
        ],
    )
    def sc_kernel(d0_hbm, d1_hbm, d2_hbm, d3_hbm, pcb_hbm, ct_hbm, tbl_hbm,
                  d2rs_hbm, out_hbm, difb, pcbb, ctb, tblb, d2b, rsb, outb,
                  wvb, idxb, sem_a, sem_b):
        wid = lax.axis_index("s") * 2 + lax.axis_index("c")
        base = wid * _RPW
        bat = base // _L
        gbase = bat * _L
        rloc0 = base - gbase
        jidx = lax.iota(jnp.int32, _LANES)
        sel10 = jidx < _K

        pend_a = []
        for q, src in enumerate((d0_hbm, d1_hbm, d2_hbm, d3_hbm)):
            pend_a.append(pltpu.async_copy(
                src.at[pl.ds(base * _C, _QW)],
                difb.at[pl.ds(q * _QW, _QW)], sem_a))
        pend_a.append(pltpu.async_copy(
            d2rs_hbm.at[pl.ds(bat * 2 * _L, _L)], d2b, sem_a))
        pend_a.append(pltpu.async_copy(
            d2rs_hbm.at[pl.ds(bat * 2 * _L + _L, _L)], rsb, sem_a))
        pend_b = [pltpu.async_copy(
            pcb_hbm.at[pl.ds(base * _OUT, _RPW * _OUT)], pcbb, sem_b)]
        pend_b.append(pltpu.async_copy(
            ct_hbm.at[pl.ds(base * _OUT, _RPW * _OUT)], ctb, sem_b))
        pend_b.append(pltpu.async_copy(tbl_hbm.at[pl.ds(gbase, _L)], tblb,
                                       sem_b))
        for h in pend_a:
            h.wait()

        @plsc.parallel_loop(0, _RPW)
        def sort_body(r):
            keys, vals = _topk16(difb, r, jidx)
            idx_loc = jnp.where(sel10, vals, 0)
            d2g = plsc.load_gather(d2b, [idx_loc])
            rsg = plsc.load_gather(rsb, [idx_loc])
            d2i = plsc.load_gather(d2b, [jidx * 0 + (rloc0 + r)])
            rsi = plsc.load_gather(rsb, [jidx * 0 + (rloc0 + r)])
            wvb[pl.ds(r * _LANES, _LANES)] = (
                (d2i + d2g - keys) * 0.5 * rsi * rsg)
            idxb[pl.ds(r * _LANES, _LANES)] = idx_loc

        for h in pend_b:
            h.wait()

        @plsc.parallel_loop(0, _RPW // 2)
        def pair_body(rp):
            acts = [[None] * 2 for _ in range(_OUT // _LANES)]
            for h in range(2):
                r = 2 * rp + h
                wv = wvb[pl.ds(r * _LANES, _LANES)]
                idx_loc = idxb[pl.ds(r * _LANES, _LANES)]
                bvecs = [jidx * 0 + idx_loc[s] for s in range(_K)]
                for c in range(_OUT // _LANES):
                    cvec = jidx + c * _LANES
                    pcb_c = pcbb[pl.ds(r * _OUT + c * _LANES, _LANES)]
                    acc = jnp.full((_LANES,), -3.0e38, jnp.float32)
                    for s in range(_K):
                        row_s = plsc.load_gather(tblb, [bvecs[s], cvec])
                        acc = jnp.maximum(acc, (row_s + pcb_c) * wv[s])
                    ct_c = ctb[pl.ds(r * _OUT + c * _LANES, _LANES)]
                    acts[c][h] = jnp.maximum(acc + ct_c, 0.0)
            for c in range(_OUT // _LANES):
                outb[pl.ds(rp * _OUT + c * _LANES, _LANES)] = (
                    jnp.maximum(acts[c][0], acts[c][1]))

        pltpu.sync_copy(
            outb, out_hbm.at[pl.ds(base // 2 * _OUT, _RPW // 2 * _OUT)])

    return sc_kernel(d0, d1, d2f, d3, pcbf, ctf, tbl, d2rs_f)


def kernel(x, num_frms, Wc, bc, Wl, bl):
    del num_frms
    bs = x.shape[0]
    w0 = Wc[:, :, 0]
    w1 = Wc[:, :, 1]
    w2 = Wc[:, :, 2]
    wln = Wl[:, :_C]
    wlc = Wl[:, _C:]
    d0, d1, d2f, d3, pcb, ct, tbl, d2rs = _tc_stage(
        x, w0, w1, w2, bc.reshape(1, _OUT), wln, wlc, bl.reshape(1, _OUT))
    flat = lambda a: a.reshape(bs * _L * _C)
    pooled = _sc_stage(flat(d0), flat(d1), flat(d2f), flat(d3),
                       flat(pcb), flat(ct),
                       tbl.reshape(bs * _L, _OUT),
                       d2rs.reshape(bs * 2 * _L))
    return jnp.transpose(pooled.reshape(bs, _L // 2, _OUT), (0, 2, 1))


# speedup vs baseline: 1.1148x; 1.1148x over previous
import functools

import jax
import jax.numpy as jnp
from jax import lax
from jax.experimental import pallas as pl
from jax.experimental.pallas import tpu as pltpu
from jax.experimental.pallas import tpu_sc as plsc

_L = 512
_C = 128
_OUT = 128
_K = 10
_NW = 32
_RPW = 4 * _L // _NW
_LANES = 16
_NCH = _L // _LANES
_QW = _RPW * _C


def _tc_body(x_ref, w0_ref, w1_ref, w2_ref, bc_ref, wln_ref, wlc_ref, bl_ref,
             d0_ref, d1_ref, d2_ref_o, d3_ref, pcb_ref, ct_ref, tbl_ref,
             d2rs_ref):
    xb = x_ref[0]
    hi = jax.lax.Precision.HIGHEST
    dg = functools.partial(jax.lax.dot_general, precision=hi,
                           preferred_element_type=jnp.float32)

    G = dg(xb, xb, (((0,), (0,)), ((), ())))
    d2 = jnp.sum(xb * xb, axis=0)
    rs = jax.lax.rsqrt(d2)
    dif = d2[None, :] + d2[:, None] - 2.0 * G
    d0_ref[0] = dif[:, 0 * _C:1 * _C]
    d1_ref[0] = dif[:, 1 * _C:2 * _C]
    d2_ref_o[0] = dif[:, 2 * _C:3 * _C]
    d3_ref[0] = dif[:, 3 * _C:4 * _C]
    d2rs_ref[0, 0] = jnp.concatenate([d2, rs], axis=0)

    zcol = jnp.zeros((_C, 1), jnp.float32)
    xl = jnp.concatenate([xb[:, 1:], zcol], axis=1)
    xr = jnp.concatenate([zcol, xb[:, :-1]], axis=1)
    ct_ref[0] = (dg(xr, w0_ref[...], (((0,), (1,)), ((), ())))
                 + dg(xb, w1_ref[...], (((0,), (1,)), ((), ())))
                 + dg(xl, w2_ref[...], (((0,), (1,)), ((), ())))
                 + bc_ref[...])

    tbl_ref[0] = dg(xb, wln_ref[...], (((0,), (1,)), ((), ())))
    pcb_ref[0] = dg(xb, wlc_ref[...], (((0,), (1,)), ((), ()))) + bl_ref[...]


def _tc_stage(x, w0, w1, w2, bc, wln, wlc, bl):
    bs = x.shape[0]
    full = lambda s: pl.BlockSpec(s, lambda b: (0,) * len(s))
    plane = pl.BlockSpec((1, _L, _C), lambda b: (b, 0, 0))
    pshape = jax.ShapeDtypeStruct((bs, _L, _C), jnp.float32)
    return pl.pallas_call(
        _tc_body,
        grid=(bs,),
        in_specs=[
            pl.BlockSpec((1, _C, _L), lambda b: (b, 0, 0)),
            full((_OUT, _C)), full((_OUT, _C)), full((_OUT, _C)),
            full((1, _OUT)),
            full((_OUT, _C)), full((_OUT, _C)),
            full((1, _OUT)),
        ],
        out_specs=[plane] * 7 + [pl.BlockSpec((1, 1, 2 * _L),
                                              lambda b: (b, 0, 0))],
        out_shape=[pshape] * 7 + [jax.ShapeDtypeStruct((bs, 1, 2 * _L),
                                                       jnp.float32)],
    )(x, w0, w1, w2, bc, wln, wlc, bl)


def _topk16(difb, r, jidx):
    def merge(a, b, desc):
        ka, va = a
        kb, vb = b
        take = ka <= kb
        mk = jnp.where(take, ka, kb)
        mv = jnp.where(take, va, vb)
        return plsc.sort_key_val(mk, mv, descending=desc)

    def tree(lo, hi, desc):
        if hi - lo == 1:
            off = (lo // 8) * _QW + r * _C + (lo % 8) * _LANES
            k = difb[pl.ds(off, _LANES)]
            return plsc.sort_key_val(k, jidx + lo * _LANES, descending=desc)
        mid = (lo + hi) // 2
        return merge(tree(lo, mid, False), tree(mid, hi, True), desc)

    return tree(0, _NCH, False)


def _sc_stage(d0, d1, d2f, d3, pcbf, ctf, tbl, d2rs_f):
    mesh = plsc.VectorSubcoreMesh(core_axis_name="c", subcore_axis_name="s")

    @functools.partial(
        pl.kernel,
        mesh=mesh,
        compiler_params=pltpu.CompilerParams(needs_layout_passes=False),
        out_type=jax.ShapeDtypeStruct((4 * _L // 2 * _OUT,), jnp.float32),
        scratch_types=[
            pltpu.VMEM((4 * _QW,), jnp.float32),
            pltpu.VMEM((_RPW * _OUT,), jnp.float32),
            pltpu.VMEM((_RPW * _OUT,), jnp.float32),
            pltpu.VMEM((_L, _OUT), jnp.float32),
            pltpu.VMEM((_L,), jnp.float32),
            pltpu.VMEM((_L,), jnp.float32),
            pltpu.VMEM((_RPW // 2 * _OUT,), jnp.float32),
            pltpu.VMEM((_RPW * _LANES,), jnp.float32),
            pltpu.VMEM((_RPW * _LANES,), jnp.int32),
            pltpu.SemaphoreType.DMA,
            pltpu.SemaphoreType.DMA,
        ],
    )
    def sc_kernel(d0_hbm, d1_hbm, d2_hbm, d3_hbm, pcb_hbm, ct_hbm, tbl_hbm,
                  d2rs_hbm, out_hbm, difb, pcbb, ctb, tblb, d2b, rsb, outb,
                  wvb, idxb, sem_a, sem_b):
        wid = lax.axis_index("s") * 2 + lax.axis_index("c")
        base = wid * _RPW
        bat = base // _L
        gbase = bat * _L
        rloc0 = base - gbase
        jidx = lax.iota(jnp.int32, _LANES)
        sel10 = jidx < _K

        pend_a = []
        for q, src in enumerate((d0_hbm, d1_hbm, d2_hbm, d3_hbm)):
            pend_a.append(pltpu.async_copy(
                src.at[pl.ds(base * _C, _QW)],
                difb.at[pl.ds(q * _QW, _QW)], sem_a))
        pend_a.append(pltpu.async_copy(
            d2rs_hbm.at[pl.ds(bat * 2 * _L, _L)], d2b, sem_a))
        pend_a.append(pltpu.async_copy(
            d2rs_hbm.at[pl.ds(bat * 2 * _L + _L, _L)], rsb, sem_a))
        pend_b = [pltpu.async_copy(
            pcb_hbm.at[pl.ds(base * _OUT, _RPW * _OUT)], pcbb, sem_b)]
        pend_b.append(pltpu.async_copy(
            ct_hbm.at[pl.ds(base * _OUT, _RPW * _OUT)], ctb, sem_b))
        pend_b.append(pltpu.async_copy(tbl_hbm.at[pl.ds(gbase, _L)], tblb,
                                       sem_b))
        for h in pend_a:
            h.wait()

        def sort_body(r, _):
            keys, vals = _topk16(difb, r, jidx)
            idx_loc = jnp.where(sel10, vals, 0)
            d2g = plsc.load_gather(d2b, [idx_loc])
            rsg = plsc.load_gather(rsb, [idx_loc])
            d2i = plsc.load_gather(d2b, [jidx * 0 + (rloc0 + r)])
            rsi = plsc.load_gather(rsb, [jidx * 0 + (rloc0 + r)])
            wvb[pl.ds(r * _LANES, _LANES)] = (
                (d2i + d2g - keys) * 0.5 * rsi * rsg)
            idxb[pl.ds(r * _LANES, _LANES)] = idx_loc
            return ()

        lax.fori_loop(0, _RPW, sort_body, ())
        for h in pend_b:
            h.wait()

        def pair_body(rp, _):
            acts = [[None] * 2 for _ in range(_OUT // _LANES)]
            for h in range(2):
                r = 2 * rp + h
                wv = wvb[pl.ds(r * _LANES, _LANES)]
                idx_loc = idxb[pl.ds(r * _LANES, _LANES)]
                bvecs = [jidx * 0 + idx_loc[s] for s in range(_K)]
                for c in range(_OUT // _LANES):
                    cvec = jidx + c * _LANES
                    pcb_c = pcbb[pl.ds(r * _OUT + c * _LANES, _LANES)]
                    acc = jnp.full((_LANES,), -3.0e38, jnp.float32)
                    for s in range(_K):
                        row_s = plsc.load_gather(tblb, [bvecs[s], cvec])
                        acc = jnp.maximum(acc, (row_s + pcb_c) * wv[s])
                    ct_c = ctb[pl.ds(r * _OUT + c * _LANES, _LANES)]
                    acts[c][h] = jnp.maximum(acc + ct_c, 0.0)
            for c in range(_OUT // _LANES):
                outb[pl.ds(rp * _OUT + c * _LANES, _LANES)] = (
                    jnp.maximum(acts[c][0], acts[c][1]))
            return ()

        lax.fori_loop(0, _RPW // 2, pair_body, ())
        pltpu.sync_copy(
            outb, out_hbm.at[pl.ds(base // 2 * _OUT, _RPW // 2 * _OUT)])

    return sc_kernel(d0, d1, d2f, d3, pcbf, ctf, tbl, d2rs_f)


def kernel(x, num_frms, Wc, bc, Wl, bl):
    del num_frms
    bs = x.shape[0]
    w0 = Wc[:, :, 0]
    w1 = Wc[:, :, 1]
    w2 = Wc[:, :, 2]
    wln = Wl[:, :_C]
    wlc = Wl[:, _C:]
    d0, d1, d2f, d3, pcb, ct, tbl, d2rs = _tc_stage(
        x, w0, w1, w2, bc.reshape(1, _OUT), wln, wlc, bl.reshape(1, _OUT))
    flat = lambda a: a.reshape(bs * _L * _C)
    pooled = _sc_stage(flat(d0), flat(d1), flat(d2f), flat(d3),
                       flat(pcb), flat(ct),
                       tbl.reshape(bs * _L, _OUT),
                       d2rs.reshape(bs * 2 * _L))
    return jnp.transpose(pooled.reshape(bs, _L // 2, _OUT), (0, 2, 1))
